# baseline (speedup 1.0000x reference)
import functools

import jax
import jax.numpy as jnp
from jax import lax
from jax.experimental import pallas as pl
from jax.experimental.pallas import tpu as pltpu
from jax.experimental.pallas import tpu_sc as plsc

_B, _L, _V, _D = 1024, 1024, 21, 128
_EPS = 1e-5


_LB = 256


def _precompute_body(pe_ref, table_ref, gamma_ref, beta_ref, out_ref):
    pe_t = pe_ref[...]
    g = gamma_ref[...]
    b = beta_ref[...]
    for v in range(_V):
        x = pe_t + table_ref[pl.ds(v, 1), :]
        mean = jnp.mean(x, axis=1, keepdims=True)
        var = jnp.mean((x - mean) ** 2, axis=1, keepdims=True)
        y = (x - mean) / jnp.sqrt(var + _EPS)
        out_ref[:, v, :] = y * g + b


def _precompute(pe, table, gamma, beta):
    return pl.pallas_call(
        _precompute_body,
        grid=(_L // _LB,),
        in_specs=[
            pl.BlockSpec((_LB, _D), lambda i: (i, 0)),
            pl.BlockSpec((_V, _D), lambda i: (0, 0)),
            pl.BlockSpec((1, _D), lambda i: (0, 0)),
            pl.BlockSpec((1, _D), lambda i: (0, 0)),
        ],
        out_specs=pl.BlockSpec((_LB, _V, _D), lambda i: (i, 0, 0)),
        out_shape=jax.ShapeDtypeStruct((_L, _V, _D), jnp.float32),
    )(pe, table, gamma.reshape(1, _D), beta.reshape(1, _D))


_NC, _NS = 2, 16
_LH = _L // _NC
_HROWS = _LH * _V
_BPW = _B // _NS
_CHUNK = 64
_RING = 4
_GSEQ = 4
_CPS = _LH // _CHUNK
_GCH = _GSEQ * _CPS


@functools.cache
def _make_gather():
    mesh = plsc.VectorSubcoreMesh(core_axis_name="c", subcore_axis_name="s")

    @functools.partial(
        pl.kernel,
        mesh=mesh,
        out_type=jax.ShapeDtypeStruct((_B * _L // _CHUNK, _CHUNK, _D), jnp.float32),
        scratch_types=[
            pltpu.VMEM_SHARED((_HROWS, _D), jnp.float32),
            pltpu.VMEM((_GSEQ, _LH), jnp.int32),
            pltpu.VMEM((_LH,), jnp.int32),
            pltpu.VMEM((_GSEQ * _LH,), jnp.int32),
        ] + [pltpu.VMEM((_CHUNK, _D), jnp.float32) for _ in range(_RING)] + [
            pltpu.SemaphoreType.DMA,
            pltpu.SemaphoreType.DMA,
        ],
    )
    def gather_k(tok_hbm, loff_hbm, precomp_hbm, out_hbm,
                 shared_v, tok_v, loff_v, idx_v, *rest):
        rows = rest[:_RING]
        semg, semw = rest[_RING], rest[_RING + 1]
        c = lax.axis_index("c")
        s = lax.axis_index("s")
        rows_per_sub = _HROWS // _NS
        pltpu.sync_copy(
            precomp_hbm.at[pl.ds(c * _HROWS + s * rows_per_sub, rows_per_sub)],
            shared_v.at[pl.ds(s * rows_per_sub, rows_per_sub)])
        pltpu.sync_copy(loff_hbm, loff_v)
        plsc.subcore_barrier()

        def pending_write(buf):
            return pltpu.make_async_copy(buf, out_hbm.at[0], semw)

        def group_body(g, carry):
            b0 = s * _BPW + g * _GSEQ
            pltpu.sync_copy(
                tok_hbm.at[pl.ds(b0, _GSEQ), pl.ds(c * _LH, _LH)], tok_v)
            for r in range(_GSEQ):
                for q in range(_LH // 16):
                    sl = pl.ds(q * 16, 16)
                    idx_v[pl.ds(r * _LH + q * 16, 16)] = tok_v[r, sl] + loff_v[sl]

            @pl.when(g > 0)
            def _():
                for p in range(_RING - 1):
                    pending_write(rows[p]).wait()

            gcp = [None] * _RING
            wcp = [None] * _RING
            for t in range(_GCH):
                p = t % _RING
                if t >= _RING:
                    wcp[p].wait()
                gcp[p] = pltpu.async_copy(
                    shared_v.at[idx_v.at[pl.ds(t * _CHUNK, _CHUNK)]],
                    rows[p], semg)
                if t >= 1:
                    q = (t - 1) % _RING
                    r, j = divmod(t - 1, _CPS)
                    gcp[q].wait()
                    wcp[q] = pltpu.async_copy(
                        rows[q], out_hbm.at[(b0 + r) * (_L // _CHUNK) + c * _CPS + j],
                        semw)
            p = (_GCH - 1) % _RING
            r, j = divmod(_GCH - 1, _CPS)
            gcp[p].wait()
            wcp[p] = pltpu.async_copy(
                rows[p], out_hbm.at[(b0 + r) * (_L // _CHUNK) + c * _CPS + j],
                semw)
            return carry

        lax.fori_loop(0, _BPW // _GSEQ, group_body, 0)
        for p in range(_RING - 1):
            pending_write(rows[p]).wait()

    return gather_k


def kernel(seq_tokens, table, pe, gamma, beta):
    precomp = _precompute(pe, table, gamma, beta)
    precomp_flat = precomp.reshape(_L * _V, _D)
    loff = jnp.arange(_LH, dtype=jnp.int32) * _V
    out = _make_gather()(seq_tokens, loff, precomp_flat)
    return out.reshape(_B, _L, _D)


# speedup vs baseline: 1.0062x; 1.0062x over previous
import functools

import jax
import jax.numpy as jnp
from jax import lax
from jax.experimental import pallas as pl
from jax.experimental.pallas import tpu as pltpu
from jax.experimental.pallas import tpu_sc as plsc

_B, _L, _V, _D = 1024, 1024, 21, 128
_EPS = 1e-5


_LB = 256


def _precompute_body(pe_ref, table_ref, gamma_ref, beta_ref, out_ref):
    pe_t = pe_ref[...]
    g = gamma_ref[...]
    b = beta_ref[...]
    for v in range(_V):
        x = pe_t + table_ref[pl.ds(v, 1), :]
        mean = jnp.mean(x, axis=1, keepdims=True)
        var = jnp.mean((x - mean) ** 2, axis=1, keepdims=True)
        y = (x - mean) / jnp.sqrt(var + _EPS)
        out_ref[:, v, :] = y * g + b


def _precompute(pe, table, gamma, beta):
    return pl.pallas_call(
        _precompute_body,
        grid=(_L // _LB,),
        in_specs=[
            pl.BlockSpec((_LB, _D), lambda i: (i, 0)),
            pl.BlockSpec((_V, _D), lambda i: (0, 0)),
            pl.BlockSpec((1, _D), lambda i: (0, 0)),
            pl.BlockSpec((1, _D), lambda i: (0, 0)),
        ],
        out_specs=pl.BlockSpec((_LB, _V, _D), lambda i: (i, 0, 0)),
        out_shape=jax.ShapeDtypeStruct((_L, _V, _D), jnp.float32),
    )(pe, table, gamma.reshape(1, _D), beta.reshape(1, _D))


_NC, _NS = 2, 16
_LH = _L // _NC
_HROWS = _LH * _V
_BPW = _B // _NS
_CHUNK = 128
_GSEQ = 8
_CPS = _LH // _CHUNK
_GCH = _GSEQ * _CPS


@functools.cache
def _make_gather():
    mesh = plsc.VectorSubcoreMesh(core_axis_name="c", subcore_axis_name="s")

    @functools.partial(
        pl.kernel,
        mesh=mesh,
        out_type=jax.ShapeDtypeStruct((_B * _L // _CHUNK, _CHUNK, _D), jnp.float32),
        scratch_types=[
            pltpu.VMEM_SHARED((_HROWS, _D), jnp.float32),
            pltpu.VMEM((_GSEQ, _LH), jnp.int32),
            pltpu.VMEM((_LH,), jnp.int32),
            pltpu.VMEM((_GSEQ * _LH,), jnp.int32),
            pltpu.VMEM((_CHUNK, _D), jnp.float32),
            pltpu.VMEM((_CHUNK, _D), jnp.float32),
            pltpu.SemaphoreType.DMA,
            pltpu.SemaphoreType.DMA,
        ],
    )
    def gather_k(tok_hbm, loff_hbm, precomp_hbm, out_hbm,
                 shared_v, tok_v, loff_v, idx_v, rows_a, rows_b, semg, semw):
        c = lax.axis_index("c")
        s = lax.axis_index("s")
        rows_per_sub = _HROWS // _NS
        pltpu.sync_copy(
            precomp_hbm.at[pl.ds(c * _HROWS + s * rows_per_sub, rows_per_sub)],
            shared_v.at[pl.ds(s * rows_per_sub, rows_per_sub)])
        pltpu.sync_copy(loff_hbm, loff_v)
        plsc.subcore_barrier()

        rows = (rows_a, rows_b)

        def pending_write(buf):
            return pltpu.make_async_copy(buf, out_hbm.at[0], semw)

        def group_body(g, carry):
            b0 = s * _BPW + g * _GSEQ
            pltpu.sync_copy(
                tok_hbm.at[pl.ds(b0, _GSEQ), pl.ds(c * _LH, _LH)], tok_v)
            for r in range(_GSEQ):
                for q in range(_LH // 16):
                    sl = pl.ds(q * 16, 16)
                    idx_v[pl.ds(r * _LH + q * 16, 16)] = tok_v[r, sl] + loff_v[sl]

            @pl.when(g > 0)
            def _():
                pending_write(rows_a).wait()
                pending_write(rows_b).wait()

            gcp = [None, None]
            wcp = [None, None]
            for t in range(_GCH):
                p = t % 2
                if t >= 2:
                    wcp[p].wait()
                gcp[p] = pltpu.async_copy(
                    shared_v.at[idx_v.at[pl.ds(t * _CHUNK, _CHUNK)]],
                    rows[p], semg)
                if t >= 1:
                    q = (t - 1) % 2
                    r, j = divmod(t - 1, _CPS)
                    gcp[q].wait()
                    wcp[q] = pltpu.async_copy(
                        rows[q], out_hbm.at[(b0 + r) * (_L // _CHUNK) + c * _CPS + j],
                        semw)
            p = (_GCH - 1) % 2
            r, j = divmod(_GCH - 1, _CPS)
            gcp[p].wait()
            wcp[p] = pltpu.async_copy(
                rows[p], out_hbm.at[(b0 + r) * (_L // _CHUNK) + c * _CPS + j],
                semw)
            return carry

        lax.fori_loop(0, _BPW // _GSEQ, group_body, 0)
        pending_write(rows_a).wait()
        pending_write(rows_b).wait()

    return gather_k


def kernel(seq_tokens, table, pe, gamma, beta):
    precomp = _precompute(pe, table, gamma, beta)
    precomp_flat = precomp.reshape(_L * _V, _D)
    loff = jnp.arange(_LH, dtype=jnp.int32) * _V
    out = _make_gather()(seq_tokens, loff, precomp_flat)
    return out.reshape(_B, _L, _D)


# speedup vs baseline: 1.0474x; 1.0410x over previous
import functools

import jax
import jax.numpy as jnp
from jax import lax
from jax.experimental import pallas as pl
from jax.experimental.pallas import tpu as pltpu
from jax.experimental.pallas import tpu_sc as plsc

_B, _L, _V, _D = 1024, 1024, 21, 128
_VP = 24
_EPS = 1e-5


_LB = 256


def _precompute_body(pe_ref, table_ref, gamma_ref, beta_ref, out_ref):
    pe_t = pe_ref[...]
    g = gamma_ref[...]
    b = beta_ref[...]
    for v in range(_V):
        x = pe_t + table_ref[pl.ds(v, 1), :]
        mean = jnp.mean(x, axis=1, keepdims=True)
        var = jnp.mean((x - mean) ** 2, axis=1, keepdims=True)
        y = (x - mean) / jnp.sqrt(var + _EPS)
        out_ref[:, v, :] = y * g + b


def _precompute(pe, table, gamma, beta):
    return pl.pallas_call(
        _precompute_body,
        grid=(_L // _LB,),
        in_specs=[
            pl.BlockSpec((_LB, _D), lambda i: (i, 0)),
            pl.BlockSpec((_V, _D), lambda i: (0, 0)),
            pl.BlockSpec((1, _D), lambda i: (0, 0)),
            pl.BlockSpec((1, _D), lambda i: (0, 0)),
        ],
        out_specs=pl.BlockSpec((_LB, _VP, _D), lambda i: (i, 0, 0)),
        out_shape=jax.ShapeDtypeStruct((_L, _VP, _D), jnp.float32),
    )(pe, table, gamma.reshape(1, _D), beta.reshape(1, _D))


_NC, _NS = 2, 16
_LH = _L // _NC
_HROWS = _LH * _VP
_BPW = _B // _NS
_CHUNK = 64
_RING = 3
_GSEQ = 4
_CPS = _LH // _CHUNK
_GCH = _GSEQ * _CPS
_PPC = _CHUNK // 16


@functools.cache
def _make_gather():
    mesh = plsc.VectorSubcoreMesh(core_axis_name="c", subcore_axis_name="s")

    @functools.partial(
        pl.kernel,
        mesh=mesh,
        out_type=jax.ShapeDtypeStruct((_B * _L // _CHUNK, _CHUNK, _D), jnp.float32),
        scratch_types=[
            pltpu.VMEM_SHARED((_HROWS, _D), jnp.float32),
            pltpu.VMEM((_GSEQ, _LH), jnp.int32),
            pltpu.VMEM((_LH,), jnp.int32),
            pltpu.VMEM((_GSEQ * _LH,), jnp.int32),
        ] + [pltpu.VMEM((_CHUNK, _D), jnp.float32) for _ in range(_RING)] + [
            pltpu.SemaphoreType.DMA,
            pltpu.SemaphoreType.DMA,
        ],
    )
    def gather_k(tok_hbm, loff_hbm, precomp_hbm, out_hbm,
                 shared_v, tok_v, loff_v, idx_v, *rest):
        rows = rest[:_RING]
        semg, semw = rest[_RING], rest[_RING + 1]
        c = lax.axis_index("c")
        s = lax.axis_index("s")
        rows_per_sub = _HROWS // _NS
        pltpu.sync_copy(
            precomp_hbm.at[pl.ds(c * _HROWS + s * rows_per_sub, rows_per_sub)],
            shared_v.at[pl.ds(s * rows_per_sub, rows_per_sub)])
        pltpu.sync_copy(loff_hbm, loff_v)
        plsc.subcore_barrier()

        def pending_write(buf):
            return pltpu.make_async_copy(buf, out_hbm.at[0], semw)

        def group_body(g, carry):
            b0 = s * _BPW + g * _GSEQ
            pltpu.sync_copy(
                tok_hbm.at[pl.ds(b0, _GSEQ), pl.ds(c * _LH, _LH)], tok_v)

            @pl.when(g > 0)
            def _():
                for p in range(_RING):
                    pending_write(rows[p]).wait()

            gcp = [None] * _RING
            wcp = [None] * _RING
            for t in range(_GCH):
                p = t % _RING
                r, j = divmod(t, _CPS)
                for m in range(_PPC):
                    col = j * _CHUNK + m * 16
                    idx_v[pl.ds(r * _LH + col, 16)] = (
                        tok_v[r, pl.ds(col, 16)] + loff_v[pl.ds(col, 16)])
                if t >= _RING:
                    wcp[p].wait()
                gcp[p] = pltpu.async_copy(
                    shared_v.at[idx_v.at[pl.ds(t * _CHUNK, _CHUNK)]],
                    rows[p], semg)
                if t >= 1:
                    q = (t - 1) % _RING
                    r, j = divmod(t - 1, _CPS)
                    gcp[q].wait()
                    wcp[q] = pltpu.async_copy(
                        rows[q],
                        out_hbm.at[(b0 + r) * (_L // _CHUNK) + c * _CPS + j],
                        semw)
            p = (_GCH - 1) % _RING
            r, j = divmod(_GCH - 1, _CPS)
            gcp[p].wait()
            wcp[p] = pltpu.async_copy(
                rows[p], out_hbm.at[(b0 + r) * (_L // _CHUNK) + c * _CPS + j],
                semw)
            return carry

        lax.fori_loop(0, _BPW // _GSEQ, group_body, 0)
        for p in range(_RING):
            pending_write(rows[p]).wait()

    return gather_k


def kernel(seq_tokens, table, pe, gamma, beta):
    precomp = _precompute(pe, table, gamma, beta)
    precomp_flat = precomp.reshape(_L * _VP, _D)
    loff = jnp.arange(_LH, dtype=jnp.int32) * _VP
    out = _make_gather()(seq_tokens, loff, precomp_flat)
    return out.reshape(_B, _L, _D)


# speedup vs baseline: 1.0784x; 1.0296x over previous
import functools

import jax
import jax.numpy as jnp
from jax import lax
from jax.experimental import pallas as pl
from jax.experimental.pallas import tpu as pltpu
from jax.experimental.pallas import tpu_sc as plsc

_B, _L, _V, _D = 1024, 1024, 21, 128
_VP = 24
_EPS = 1e-5


_LB = 256


def _precompute_body(pe_ref, table_ref, gamma_ref, beta_ref, out_ref):
    pe_t = pe_ref[...]
    tab = table_ref[...]
    g = gamma_ref[...]
    b = beta_ref[...]
    cpe = pe_t - jnp.mean(pe_t, axis=1, keepdims=True)
    ct = tab - jnp.mean(tab, axis=1, keepdims=True)
    vpe = jnp.mean(cpe * cpe, axis=1, keepdims=True)
    vt = jnp.mean(ct * ct, axis=1, keepdims=True)
    cross = lax.dot_general(cpe, ct, (((1,), (1,)), ((), ())),
                            preferred_element_type=jnp.float32)
    var = vpe + vt.reshape(1, _V) + (2.0 / _D) * cross
    rstd = 1.0 / jnp.sqrt(var + _EPS)
    for v in range(_V):
        y = (cpe + ct[v:v + 1, :]) * rstd[:, v:v + 1]
        out_ref[:, v, :] = y * g + b


def _precompute(pe, table, gamma, beta):
    return pl.pallas_call(
        _precompute_body,
        grid=(_L // _LB,),
        in_specs=[
            pl.BlockSpec((_LB, _D), lambda i: (i, 0)),
            pl.BlockSpec((_V, _D), lambda i: (0, 0)),
            pl.BlockSpec((1, _D), lambda i: (0, 0)),
            pl.BlockSpec((1, _D), lambda i: (0, 0)),
        ],
        out_specs=pl.BlockSpec((_LB, _VP, _D), lambda i: (i, 0, 0)),
        out_shape=jax.ShapeDtypeStruct((_L, _VP, _D), jnp.float32),
    )(pe, table, gamma.reshape(1, _D), beta.reshape(1, _D))


_NC, _NS = 2, 16
_LH = _L // _NC
_HROWS = _LH * _VP
_BPW = _B // _NS
_CHUNK = 64
_RING = 3
_GSEQ = 4
_CPS = _LH // _CHUNK
_GCH = _GSEQ * _CPS
_PPC = _CHUNK // 16


@functools.cache
def _make_gather():
    mesh = plsc.VectorSubcoreMesh(core_axis_name="c", subcore_axis_name="s")

    @functools.partial(
        pl.kernel,
        mesh=mesh,
        out_type=jax.ShapeDtypeStruct((_B * _L // _CHUNK, _CHUNK, _D), jnp.float32),
        scratch_types=[
            pltpu.VMEM_SHARED((_HROWS, _D), jnp.float32),
            pltpu.VMEM((_GSEQ, _LH), jnp.int32),
            pltpu.VMEM((_LH,), jnp.int32),
            pltpu.VMEM((_GSEQ * _LH,), jnp.int32),
        ] + [pltpu.VMEM((_CHUNK, _D), jnp.float32) for _ in range(_RING)] + [
            pltpu.SemaphoreType.DMA,
            pltpu.SemaphoreType.DMA,
        ],
    )
    def gather_k(tok_hbm, loff_hbm, precomp_hbm, out_hbm,
                 shared_v, tok_v, loff_v, idx_v, *rest):
        rows = rest[:_RING]
        semg, semw = rest[_RING], rest[_RING + 1]
        c = lax.axis_index("c")
        s = lax.axis_index("s")
        rows_per_sub = _HROWS // _NS
        pltpu.sync_copy(
            precomp_hbm.at[pl.ds(c * _HROWS + s * rows_per_sub, rows_per_sub)],
            shared_v.at[pl.ds(s * rows_per_sub, rows_per_sub)])
        pltpu.sync_copy(loff_hbm, loff_v)
        plsc.subcore_barrier()

        def pending_write(buf):
            return pltpu.make_async_copy(buf, out_hbm.at[0], semw)

        def group_body(g, carry):
            b0 = s * _BPW + g * _GSEQ
            pltpu.sync_copy(
                tok_hbm.at[pl.ds(b0, _GSEQ), pl.ds(c * _LH, _LH)], tok_v)

            @pl.when(g > 0)
            def _():
                for p in range(_RING):
                    pending_write(rows[p]).wait()

            gcp = [None] * _RING
            wcp = [None] * _RING
            for t in range(_GCH):
                p = t % _RING
                r, j = divmod(t, _CPS)
                for m in range(_PPC):
                    col = j * _CHUNK + m * 16
                    idx_v[pl.ds(r * _LH + col, 16)] = (
                        tok_v[r, pl.ds(col, 16)] + loff_v[pl.ds(col, 16)])
                if t >= _RING:
                    wcp[p].wait()
                gcp[p] = pltpu.async_copy(
                    shared_v.at[idx_v.at[pl.ds(t * _CHUNK, _CHUNK)]],
                    rows[p], semg)
                if t >= 1:
                    q = (t - 1) % _RING
                    r, j = divmod(t - 1, _CPS)
                    gcp[q].wait()
                    wcp[q] = pltpu.async_copy(
                        rows[q],
                        out_hbm.at[(b0 + r) * (_L // _CHUNK) + c * _CPS + j],
                        semw)
            p = (_GCH - 1) % _RING
            r, j = divmod(_GCH - 1, _CPS)
            gcp[p].wait()
            wcp[p] = pltpu.async_copy(
                rows[p], out_hbm.at[(b0 + r) * (_L // _CHUNK) + c * _CPS + j],
                semw)
            return carry

        lax.fori_loop(0, _BPW // _GSEQ, group_body, 0)
        for p in range(_RING):
            pending_write(rows[p]).wait()

    return gather_k


def kernel(seq_tokens, table, pe, gamma, beta):
    precomp = _precompute(pe, table, gamma, beta)
    precomp_flat = precomp.reshape(_L * _VP, _D)
    loff = jnp.arange(_LH, dtype=jnp.int32) * _VP
    out = _make_gather()(seq_tokens, loff, precomp_flat)
    return out.reshape(_B, _L, _D)
